# Initial kernel scaffold; baseline (speedup 1.0000x reference)
import functools

import jax
import jax.numpy as jnp
from jax import lax
from jax.experimental import pallas as pl
from jax.experimental.pallas import tpu as pltpu
from jax.experimental.pallas import tpu_sc as plsc

N = 10000
E = 320000
IN = 128
H = 64

NC = 2
NS = 16
NW = NC * NS
CHUNK = 128
CPW = 79
E_PAD = NW * CPW * CHUNK
ROWS_PT = 640
N_PAD = NS * ROWS_PT
DUMMY = N_PAD - 1
W1AGG = 80
W2AGG = 16


def _tc1_body(x_ref, wl_ref, wr_ref, b1_ref, p1_ref, r1_ref):
    x = x_ref[...]
    p1 = jnp.dot(x, wl_ref[...], preferred_element_type=jnp.float32)
    ones = jnp.ones((N, 1), jnp.float32)
    zpad = jnp.zeros((N, W1AGG - H - 1), jnp.float32)
    p1_ref[...] = jnp.concatenate([p1, ones, zpad], axis=1)
    r1_ref[...] = (
        jnp.dot(x, wr_ref[...], preferred_element_type=jnp.float32) + b1_ref[...]
    )


def _tc1(x, W1_l, W1_r, b1):
    return pl.pallas_call(
        _tc1_body,
        out_shape=[
            jax.ShapeDtypeStruct((N, W1AGG), jnp.float32),
            jax.ShapeDtypeStruct((N, H), jnp.float32),
        ],
    )(x, W1_l, W1_r, b1)


def _tc2_body(s1p_ref, r1_ref, w2l_ref, w2r_ref, b2_ref, p2_ref, aux_ref):
    s1 = s1p_ref[0, :N, :] + s1p_ref[1, :N, :]
    deg = jnp.maximum(s1[:, H:H + 1], 1.0)
    rdeg = 1.0 / deg
    h = jnp.maximum(s1[:, :H] * rdeg + r1_ref[...], 0.0)
    p2 = jnp.dot(h, w2l_ref[...], preferred_element_type=jnp.float32)
    r2b = jnp.dot(h, w2r_ref[...], preferred_element_type=jnp.float32) + b2_ref[...]
    p2_ref[...] = jnp.concatenate(
        [p2, jnp.zeros((N, W2AGG - 1), jnp.float32)], axis=1)
    aux_ref[...] = jnp.concatenate(
        [rdeg, r2b, jnp.zeros((N, 6), jnp.float32)], axis=1)


def _tc2(s1p, r1b, W2_l, W2_r, b2):
    return pl.pallas_call(
        _tc2_body,
        out_shape=[
            jax.ShapeDtypeStruct((N, W2AGG), jnp.float32),
            jax.ShapeDtypeStruct((N, 8), jnp.float32),
        ],
    )(s1p, r1b, W2_l, W2_r, b2)


def _tc3_body(s2p_ref, aux_ref, out_ref):
    s2 = s2p_ref[0, :N, 0:1] + s2p_ref[1, :N, 0:1]
    out_ref[...] = s2 * aux_ref[:, 0:1] + aux_ref[:, 1:2]


def _tc3(s2p, aux):
    return pl.pallas_call(
        _tc3_body,
        out_shape=jax.ShapeDtypeStruct((N, 1), jnp.float32),
    )(s2p, aux)


def _make_sc_agg(width):
    mesh = plsc.VectorSubcoreMesh(
        core_axis_name="c", subcore_axis_name="s", num_cores=NC, num_subcores=NS)
    nzero = ROWS_PT // CHUNK

    @functools.partial(
        pl.kernel,
        mesh=mesh,
        out_type=jax.ShapeDtypeStruct((NC, N_PAD, width), jnp.float32),
        scratch_types=[
            pltpu.VMEM((CPW, CHUNK), jnp.int32),
            pltpu.VMEM((CPW, CHUNK), jnp.int32),
            pltpu.VMEM((CHUNK, width), jnp.float32),
            pltpu.VMEM((CHUNK, width), jnp.float32),
            pltpu.VMEM_SHARED((N_PAD, width), jnp.float32),
            pltpu.SemaphoreType.DMA,
        ],
    )
    def sc_agg(table_hbm, src_hbm, dst_hbm, out_hbm, src_v, dst_v, rows_v,
               zbuf, acc, sem):
        c = lax.axis_index("c")
        s = lax.axis_index("s")
        wid = s * NC + c
        base = s * ROWS_PT

        def _zrow(i, _):
            def _zcol(j, _):
                zbuf[i, pl.ds(j * 16, 16)] = jnp.zeros((16,), jnp.float32)
                return 0
            return lax.fori_loop(0, width // 16, _zcol, 0)
        lax.fori_loop(0, CHUNK, _zrow, 0)
        for k in range(nzero):
            pltpu.sync_copy(zbuf, acc.at[pl.ds(base + k * CHUNK, CHUNK), :])
        plsc.subcore_barrier()

        pltpu.sync_copy(src_hbm.at[wid], src_v)
        pltpu.sync_copy(dst_hbm.at[wid], dst_v)

        def _edge_chunk(j, _):
            pltpu.async_copy(table_hbm.at[src_v.at[j]], rows_v, sem).wait()
            pltpu.sync_copy(rows_v, acc.at[dst_v.at[j]], add=True)
            return 0
        lax.fori_loop(0, CPW, _edge_chunk, 0)
        plsc.subcore_barrier()

        pltpu.sync_copy(acc.at[pl.ds(base, ROWS_PT), :],
                        out_hbm.at[c, pl.ds(base, ROWS_PT), :])

    return sc_agg


_sc_agg_l1 = _make_sc_agg(W1AGG)
_sc_agg_l2 = _make_sc_agg(W2AGG)


def kernel(x, edge_index, W1_l, W1_r, b1, W2_l, W2_r, b2):
    pad = E_PAD - E
    src = jnp.concatenate([edge_index[0], jnp.zeros((pad,), jnp.int32)])
    dst = jnp.concatenate([edge_index[1], jnp.full((pad,), DUMMY, jnp.int32)])
    src = src.reshape(NW, CPW, CHUNK)
    dst = dst.reshape(NW, CPW, CHUNK)

    p1ext, r1b = _tc1(x, W1_l, W1_r, b1.reshape(1, H))
    s1p = _sc_agg_l1(p1ext, src, dst)
    p2ext, aux = _tc2(s1p, r1b, W2_l, W2_r, b2.reshape(1, 1))
    s2p = _sc_agg_l2(p2ext, src, dst)
    return _tc3(s2p, aux)


# speedup vs baseline: 8.4631x; 8.4631x over previous
import functools

import jax
import jax.numpy as jnp
from jax import lax
from jax.experimental import pallas as pl
from jax.experimental.pallas import tpu as pltpu
from jax.experimental.pallas import tpu_sc as plsc

N = 10000
E = 320000
IN = 128
H = 64

NC = 2
NS = 16
NW = NC * NS
CHUNK = 128
CPW = 79
E_PAD = NW * CPW * CHUNK
ROWS_PT = 640
N_PAD = NS * ROWS_PT
DUMMY = N_PAD - 1
W1AGG = 80
W2AGG = 16


def _tc1_body(x_ref, wl_ref, wr_ref, b1_ref, p1_ref, r1_ref):
    x = x_ref[...]
    p1 = jnp.dot(x, wl_ref[...], preferred_element_type=jnp.float32)
    ones = jnp.ones((N, 1), jnp.float32)
    zpad = jnp.zeros((N, W1AGG - H - 1), jnp.float32)
    p1_ref[...] = jnp.concatenate([p1, ones, zpad], axis=1)
    r1_ref[...] = (
        jnp.dot(x, wr_ref[...], preferred_element_type=jnp.float32) + b1_ref[...]
    )


def _tc1(x, W1_l, W1_r, b1):
    return pl.pallas_call(
        _tc1_body,
        out_shape=[
            jax.ShapeDtypeStruct((N, W1AGG), jnp.float32),
            jax.ShapeDtypeStruct((N, H), jnp.float32),
        ],
    )(x, W1_l, W1_r, b1)


def _tc2_body(s1p_ref, r1_ref, w2l_ref, w2r_ref, b2_ref, p2_ref, aux_ref):
    s1 = s1p_ref[0, :N, :] + s1p_ref[1, :N, :]
    deg = jnp.maximum(s1[:, H:H + 1], 1.0)
    rdeg = 1.0 / deg
    h = jnp.maximum(s1[:, :H] * rdeg + r1_ref[...], 0.0)
    p2 = jnp.dot(h, w2l_ref[...], preferred_element_type=jnp.float32)
    r2b = jnp.dot(h, w2r_ref[...], preferred_element_type=jnp.float32) + b2_ref[...]
    p2_ref[...] = jnp.concatenate(
        [p2, jnp.zeros((N, W2AGG - 1), jnp.float32)], axis=1)
    aux_ref[...] = jnp.concatenate(
        [rdeg, r2b, jnp.zeros((N, 6), jnp.float32)], axis=1)


def _tc2(s1p, r1b, W2_l, W2_r, b2):
    return pl.pallas_call(
        _tc2_body,
        out_shape=[
            jax.ShapeDtypeStruct((N, W2AGG), jnp.float32),
            jax.ShapeDtypeStruct((N, 8), jnp.float32),
        ],
    )(s1p, r1b, W2_l, W2_r, b2)


def _tc3_body(s2p_ref, aux_ref, out_ref):
    s2 = s2p_ref[0, :N, 0:1] + s2p_ref[1, :N, 0:1]
    out_ref[...] = s2 * aux_ref[:, 0:1] + aux_ref[:, 1:2]


def _tc3(s2p, aux):
    return pl.pallas_call(
        _tc3_body,
        out_shape=jax.ShapeDtypeStruct((N, 1), jnp.float32),
    )(s2p, aux)


def _make_sc_agg(width):
    mesh = plsc.VectorSubcoreMesh(
        core_axis_name="c", subcore_axis_name="s", num_cores=NC, num_subcores=NS)
    nzero = ROWS_PT // CHUNK

    @functools.partial(
        pl.kernel,
        mesh=mesh,
        compiler_params=pltpu.CompilerParams(use_tc_tiling_on_sc=False),
        out_type=jax.ShapeDtypeStruct((NC, N_PAD, width), jnp.float32),
        scratch_types=[
            pltpu.VMEM((CPW, CHUNK), jnp.int32),
            pltpu.VMEM((CPW, CHUNK), jnp.int32),
            pltpu.VMEM((CHUNK, width), jnp.float32),
            pltpu.VMEM((CHUNK, width), jnp.float32),
            pltpu.VMEM_SHARED((N_PAD, width), jnp.float32),
            pltpu.SemaphoreType.DMA,
        ],
    )
    def sc_agg(table_hbm, src_hbm, dst_hbm, out_hbm, src_v, dst_v, rows_v,
               zbuf, acc, sem):
        c = lax.axis_index("c")
        s = lax.axis_index("s")
        wid = s * NC + c
        base = s * ROWS_PT

        def _zrow(i, _):
            def _zcol(j, _):
                zbuf[i, pl.ds(j * 16, 16)] = jnp.zeros((16,), jnp.float32)
                return 0
            return lax.fori_loop(0, width // 16, _zcol, 0)
        lax.fori_loop(0, CHUNK, _zrow, 0)
        for k in range(nzero):
            pltpu.sync_copy(zbuf, acc.at[pl.ds(base + k * CHUNK, CHUNK), :])
        plsc.subcore_barrier()

        pltpu.sync_copy(src_hbm.at[wid], src_v)
        pltpu.sync_copy(dst_hbm.at[wid], dst_v)

        def _edge_chunk(j, _):
            pltpu.async_copy(table_hbm.at[src_v.at[j]], rows_v, sem).wait()
            pltpu.sync_copy(rows_v, acc.at[dst_v.at[j]], add=True)
            return 0
        lax.fori_loop(0, CPW, _edge_chunk, 0)
        plsc.subcore_barrier()

        pltpu.sync_copy(acc.at[pl.ds(base, ROWS_PT), :],
                        out_hbm.at[c, pl.ds(base, ROWS_PT), :])

    return sc_agg


_sc_agg_l1 = _make_sc_agg(W1AGG)
_sc_agg_l2 = _make_sc_agg(W2AGG)


def kernel(x, edge_index, W1_l, W1_r, b1, W2_l, W2_r, b2):
    pad = E_PAD - E
    src = jnp.concatenate([edge_index[0], jnp.zeros((pad,), jnp.int32)])
    dst = jnp.concatenate([edge_index[1], jnp.full((pad,), DUMMY, jnp.int32)])
    src = src.reshape(NW, CPW, CHUNK)
    dst = dst.reshape(NW, CPW, CHUNK)

    p1ext, r1b = _tc1(x, W1_l, W1_r, b1.reshape(1, H))
    s1p = _sc_agg_l1(p1ext, src, dst)
    p2ext, aux = _tc2(s1p, r1b, W2_l, W2_r, b2.reshape(1, 1))
    s2p = _sc_agg_l2(p2ext, src, dst)
    return _tc3(s2p, aux)
